# baseline (speedup 1.0000x reference)
import functools

import jax
import jax.numpy as jnp
from jax import lax
from jax.experimental import pallas as pl
from jax.experimental.pallas import tpu as pltpu
from jax.experimental.pallas import tpu_sc as plsc

D = 128
DH = D // 2
NC, NS = 2, 16
CH = 80


def _edge_lin_body(ea_ref, w_ref, b_ref, out_ref):
    out_ref[0] = (
        jnp.dot(ea_ref[...], w_ref[0], preferred_element_type=jnp.float32)
        + b_ref[0]
    )


def _edge_lin(ea2, W_bd, b_pair):
    E2, DE2 = ea2.shape
    BE = 1280
    return pl.pallas_call(
        _edge_lin_body,
        grid=(NC, E2 // BE),
        in_specs=[
            pl.BlockSpec((BE, DE2), lambda j, i: (i, 0)),
            pl.BlockSpec((1, DE2, D), lambda j, i: (j, 0, 0)),
            pl.BlockSpec((1, 1, D), lambda j, i: (j, 0, 0)),
        ],
        out_specs=pl.BlockSpec((1, BE, D), lambda j, i: (j, i, 0)),
        out_shape=jax.ShapeDtypeStruct((NC, E2, D), jnp.float32),
    )(ea2, W_bd, b_pair)


def _update_body(s_ref, x_ref, a_ref, w_ref, b_ref, out_ref, *, do_relu):
    h = x_ref[...] * s_ref[0] + jnp.concatenate([a_ref[0], a_ref[1]], axis=1)
    o = jnp.dot(h, w_ref[...], preferred_element_type=jnp.float32) + b_ref[...]
    if do_relu:
        o = jnp.maximum(o, 0.0)
    out_ref[...] = o


def _node_update(x, agg, W_nn, b_nn, scale, do_relu):
    N = x.shape[0]
    BN = 2000
    return pl.pallas_call(
        functools.partial(_update_body, do_relu=do_relu),
        grid=(N // BN,),
        in_specs=[
            pl.BlockSpec(memory_space=pltpu.SMEM),
            pl.BlockSpec((BN, D), lambda i: (i, 0)),
            pl.BlockSpec((NC, BN, DH), lambda i: (0, i, 0)),
            pl.BlockSpec((D, D), lambda i: (0, 0)),
            pl.BlockSpec((1, D), lambda i: (0, 0)),
        ],
        out_specs=pl.BlockSpec((BN, D), lambda i: (i, 0)),
        out_shape=jax.ShapeDtypeStruct((N, D), jnp.float32),
    )(scale, x, agg, W_nn, b_nn.reshape(1, D))


def _pad_dst(n_dst):
    per_tile = -(-n_dst // NS)
    per_tile = -(-per_tile // CH) * CH
    return NS * per_tile


def _make_sc_msg(n_src, n_dst, E):
    ept = E // NS
    nchunk = ept // CH
    n_pad = _pad_dst(n_dst)
    rows_per_tile = n_pad // NS
    n80 = rows_per_tile // CH

    @functools.partial(
        pl.kernel,
        out_type=jax.ShapeDtypeStruct((NC, n_pad, DH), jnp.float32),
        mesh=plsc.VectorSubcoreMesh(
            core_axis_name="c", subcore_axis_name="s",
            num_cores=NC, num_subcores=NS,
        ),
        scratch_types=[
            pltpu.VMEM((nchunk, CH), jnp.int32),
            pltpu.VMEM((CH,), jnp.int32),
            pltpu.VMEM((CH,), jnp.int32),
            pltpu.VMEM((CH, DH), jnp.float32),
            pltpu.VMEM((CH // 2, D), jnp.float32),
            pltpu.VMEM((CH, DH), jnp.float32),
            pltpu.VMEM((CH // 2, D), jnp.float32),
            pltpu.VMEM((CH // 2, DH), jnp.float32),
            pltpu.VMEM_SHARED((n_pad, DH), jnp.float32),
            pltpu.VMEM_SHARED((n_src, DH), jnp.float32),
            pltpu.SemaphoreType.DMA,
            pltpu.SemaphoreType.DMA,
            pltpu.SemaphoreType.DMA,
            pltpu.SemaphoreType.DMA,
            pltpu.SemaphoreType.DMA,
            pltpu.SemaphoreType.DMA,
            pltpu.SemaphoreType.DMA,
            pltpu.SemaphoreType.DMA,
        ],
        compiler_params=pltpu.CompilerParams(use_tc_tiling_on_sc=False),
    )
    def sc_msg(x_hbm, srcw_hbm, dstw_hbm, lin_hbm, out_hbm,
               sidx_v, didx_a, didx_b, rows_a, msg_a, rows_b, msg_b, zero_v,
               agg_sh, x_sh, gsem_a, lsem_a, ssem_a, dsem_a,
               gsem_b, lsem_b, ssem_b, dsem_b):
        cid = lax.axis_index("c")
        sid = lax.axis_index("s")

        zero16 = jnp.zeros((16,), jnp.float32)
        zrows = CH // 2

        def zbody(r, carry):
            for l in range(DH // 16):
                zero_v[r, pl.ds(l * 16, 16)] = zero16
            return carry

        lax.fori_loop(0, zrows, zbody, 0)
        row0 = sid * rows_per_tile
        for p in range(rows_per_tile // zrows):
            pltpu.sync_copy(zero_v.at[pl.ds(0, zrows)],
                            agg_sh.at[pl.ds(row0 + p * zrows, zrows)])

        pltpu.sync_copy(srcw_hbm.at[sid], sidx_v)
        x0 = sid * (n_src // NS)
        pltpu.sync_copy(x_hbm.at[pl.ds(x0, n_src // NS), cid],
                        x_sh.at[pl.ds(x0, n_src // NS)])
        plsc.subcore_barrier()

        base0 = sid * ept
        bufs = ((rows_a, msg_a, didx_a, gsem_a, lsem_a, ssem_a, dsem_a),
                (rows_b, msg_b, didx_b, gsem_b, lsem_b, ssem_b, dsem_b))

        def issue_loads(j, rows_r, msg_r, didx_r, gsem, lsem, dsem):
            base = pl.multiple_of((base0 + j * CH) // 2, 8)
            pltpu.async_copy(lin_hbm.at[cid, pl.ds(base, CH // 2)], msg_r,
                             lsem)
            pltpu.async_copy(x_sh.at[sidx_v.at[j]], rows_r, gsem)
            pltpu.async_copy(dstw_hbm.at[sid, j], didx_r, dsem)

        def wait_loads(rows_r, msg_r, didx_r, gsem, lsem, dsem):
            pltpu.make_async_copy(lin_hbm.at[cid, pl.ds(0, CH // 2)], msg_r,
                                  lsem).wait()
            pltpu.make_async_copy(x_sh.at[sidx_v.at[0]], rows_r,
                                  gsem).wait()
            pltpu.make_async_copy(dstw_hbm.at[sid, 0], didx_r, dsem).wait()

        def wait_scatter(rows_r, ssem):
            pltpu.make_async_copy(rows_r, agg_sh.at[didx_a], ssem).wait()

        def compute_chunk(rows_c, msg_c):
            @plsc.parallel_loop(0, CH // 2, step=1, unroll=2)
            def _(r):
                for l in range(D // 16):
                    e = 2 * r + l // (DH // 16)
                    sl = pl.ds((l % (DH // 16)) * 16, 16)
                    rows_c[e, sl] = jnp.maximum(
                        rows_c[e, sl] + msg_c[r, pl.ds(l * 16, 16)], 0.0)

        issue_loads(0, rows_a, msg_a, didx_a, gsem_a, lsem_a, dsem_a)
        issue_loads(1, rows_b, msg_b, didx_b, gsem_b, lsem_b, dsem_b)
        wait_loads(rows_a, msg_a, didx_a, gsem_a, lsem_a, dsem_a)
        compute_chunk(rows_a, msg_a)
        pltpu.async_copy(rows_a, agg_sh.at[didx_a], ssem_a, add=True)
        wait_scatter(rows_a, ssem_a)
        issue_loads(2, rows_a, msg_a, didx_a, gsem_a, lsem_a, dsem_a)
        wait_loads(rows_b, msg_b, didx_b, gsem_b, lsem_b, dsem_b)
        compute_chunk(rows_b, msg_b)
        pltpu.async_copy(rows_b, agg_sh.at[didx_b], ssem_b, add=True)

        def pair_body(p, carry):
            for b in (0, 1):
                j = 2 * p + b
                rows_c, msg_c, didx_c, gsem_c, lsem_c, ssem_c, dsem_c = \
                    bufs[b]
                rows_n, msg_n, didx_n, gsem_n, lsem_n, ssem_n, dsem_n = \
                    bufs[1 - b]
                wait_scatter(rows_n, ssem_n)
                jn = jnp.where(j + 1 < nchunk, j + 1, 0)
                issue_loads(jn, rows_n, msg_n, didx_n, gsem_n, lsem_n,
                            dsem_n)
                wait_loads(rows_c, msg_c, didx_c, gsem_c, lsem_c, dsem_c)
                compute_chunk(rows_c, msg_c)
                pltpu.async_copy(rows_c, agg_sh.at[didx_c], ssem_c,
                                 add=True)
            return carry

        lax.fori_loop(1, nchunk // 2, pair_body, 0)
        wait_loads(rows_a, msg_a, didx_a, gsem_a, lsem_a, dsem_a)
        wait_scatter(rows_b, ssem_b)
        plsc.subcore_barrier()
        pltpu.sync_copy(agg_sh.at[pl.ds(row0, rows_per_tile)],
                        out_hbm.at[cid, pl.ds(row0, rows_per_tile)])

    return sc_msg


def kernel(x_author, x_paper, edge_index_a2p, edge_index_p2a,
           edge_attr_a2p, edge_attr_p2a, W_edge, b_edge, W_nn, b_nn, eps):
    n_a, _ = x_author.shape
    n_p, _ = x_paper.shape
    E = edge_index_a2p.shape[1]
    ept = E // NS
    nchunk = ept // CH

    def slab(ei):
        return (ei[0].reshape(NS, nchunk, CH), ei[1].reshape(NS, nchunk, CH))

    src_a2p, dst_a2p = slab(edge_index_a2p)
    src_p2a, dst_p2a = slab(edge_index_p2a)

    DE = W_edge.shape[0]
    zeros = jnp.zeros((DE, DH), jnp.float32)
    W_bd = jnp.stack([
        jnp.block([[W_edge[:, :DH], zeros], [zeros, W_edge[:, :DH]]]),
        jnp.block([[W_edge[:, DH:], zeros], [zeros, W_edge[:, DH:]]]),
    ], axis=0)
    b_pair = jnp.stack([
        jnp.concatenate([b_edge[:DH], b_edge[:DH]]),
        jnp.concatenate([b_edge[DH:], b_edge[DH:]]),
    ], axis=0).reshape(NC, 1, D)
    ea2_a2p = edge_attr_a2p.reshape(E // 2, 2 * DE)
    ea2_p2a = edge_attr_p2a.reshape(E // 2, 2 * DE)
    lin_a2p = _edge_lin(ea2_a2p, W_bd, b_pair)
    lin_p2a = _edge_lin(ea2_p2a, W_bd, b_pair)

    if n_a == n_p:
        sc_a2p = sc_p2a = _make_sc_msg(n_a, n_p, E)
    else:
        sc_a2p = _make_sc_msg(n_a, n_p, E)
        sc_p2a = _make_sc_msg(n_p, n_a, E)

    scale = jnp.reshape(1.0 + eps, (1,))

    xa, xp = x_author, x_paper
    for i in range(2):
        agg_p = sc_a2p(xa.reshape(n_a, NC, DH), src_a2p, dst_a2p, lin_a2p)
        agg_a = sc_p2a(xp.reshape(n_p, NC, DH), src_p2a, dst_p2a, lin_p2a)
        do_relu = i == 0
        new_p = _node_update(xp, agg_p, W_nn, b_nn, scale, do_relu)
        new_a = _node_update(xa, agg_a, W_nn, b_nn, scale, do_relu)
        xa, xp = new_a, new_p
    return (xa, xp)


# speedup vs baseline: 1.0029x; 1.0029x over previous
import functools

import jax
import jax.numpy as jnp
from jax import lax
from jax.experimental import pallas as pl
from jax.experimental.pallas import tpu as pltpu
from jax.experimental.pallas import tpu_sc as plsc

D = 128
DH = D // 2
NC, NS = 2, 16
CH = 80


def _edge_lin_body(ea_ref, w_ref, b_ref, out_ref):
    out_ref[0] = (
        jnp.dot(ea_ref[...], w_ref[0], preferred_element_type=jnp.float32)
        + b_ref[0]
    )


def _edge_lin(ea2, W_bd, b_pair):
    E2, DE2 = ea2.shape
    BE = 1280
    return pl.pallas_call(
        _edge_lin_body,
        grid=(NC, E2 // BE),
        in_specs=[
            pl.BlockSpec((BE, DE2), lambda j, i: (i, 0)),
            pl.BlockSpec((1, DE2, D), lambda j, i: (j, 0, 0)),
            pl.BlockSpec((1, 1, D), lambda j, i: (j, 0, 0)),
        ],
        out_specs=pl.BlockSpec((1, BE, D), lambda j, i: (j, i, 0)),
        out_shape=jax.ShapeDtypeStruct((NC, E2, D), jnp.float32),
    )(ea2, W_bd, b_pair)


def _update_body(s_ref, x_ref, a_ref, w_ref, b_ref, out_ref, *, do_relu):
    h = x_ref[...] * s_ref[0] + jnp.concatenate([a_ref[0], a_ref[1]], axis=1)
    o = jnp.dot(h, w_ref[...], preferred_element_type=jnp.float32) + b_ref[...]
    if do_relu:
        o = jnp.maximum(o, 0.0)
    out_ref[...] = o


def _node_update(x, agg, W_nn, b_nn, scale, do_relu):
    N = x.shape[0]
    BN = 2000
    return pl.pallas_call(
        functools.partial(_update_body, do_relu=do_relu),
        grid=(N // BN,),
        in_specs=[
            pl.BlockSpec(memory_space=pltpu.SMEM),
            pl.BlockSpec((BN, D), lambda i: (i, 0)),
            pl.BlockSpec((NC, BN, DH), lambda i: (0, i, 0)),
            pl.BlockSpec((D, D), lambda i: (0, 0)),
            pl.BlockSpec((1, D), lambda i: (0, 0)),
        ],
        out_specs=pl.BlockSpec((BN, D), lambda i: (i, 0)),
        out_shape=jax.ShapeDtypeStruct((N, D), jnp.float32),
    )(scale, x, agg, W_nn, b_nn.reshape(1, D))


def _pad_dst(n_dst):
    per_tile = -(-n_dst // NS)
    per_tile = -(-per_tile // CH) * CH
    return NS * per_tile


def _make_sc_msg(n_src, n_dst, E):
    ept = E // NS
    nchunk = ept // CH
    n_pad = _pad_dst(n_dst)
    rows_per_tile = n_pad // NS
    n80 = rows_per_tile // CH

    @functools.partial(
        pl.kernel,
        out_type=jax.ShapeDtypeStruct((NC, n_pad, DH), jnp.float32),
        mesh=plsc.VectorSubcoreMesh(
            core_axis_name="c", subcore_axis_name="s",
            num_cores=NC, num_subcores=NS,
        ),
        scratch_types=[
            pltpu.VMEM((nchunk, CH), jnp.int32),
            pltpu.VMEM((CH,), jnp.int32),
            pltpu.VMEM((CH,), jnp.int32),
            pltpu.VMEM((CH, DH), jnp.float32),
            pltpu.VMEM((CH, DH), jnp.float32),
            pltpu.VMEM((CH, DH), jnp.float32),
            pltpu.VMEM((CH, DH), jnp.float32),
            pltpu.VMEM((CH // 2, DH), jnp.float32),
            pltpu.VMEM_SHARED((n_pad, DH), jnp.float32),
            pltpu.VMEM_SHARED((n_src, DH), jnp.float32),
            pltpu.SemaphoreType.DMA,
            pltpu.SemaphoreType.DMA,
            pltpu.SemaphoreType.DMA,
            pltpu.SemaphoreType.DMA,
            pltpu.SemaphoreType.DMA,
            pltpu.SemaphoreType.DMA,
            pltpu.SemaphoreType.DMA,
            pltpu.SemaphoreType.DMA,
        ],
        compiler_params=pltpu.CompilerParams(use_tc_tiling_on_sc=False),
    )
    def sc_msg(x_hbm, srcw_hbm, dstw_hbm, lin_hbm, out_hbm,
               sidx_v, didx_a, didx_b, rows_a, msg_a, rows_b, msg_b, zero_v,
               agg_sh, x_sh, gsem_a, lsem_a, ssem_a, dsem_a,
               gsem_b, lsem_b, ssem_b, dsem_b):
        cid = lax.axis_index("c")
        sid = lax.axis_index("s")

        zero16 = jnp.zeros((16,), jnp.float32)
        zrows = CH // 2

        def zbody(r, carry):
            for l in range(DH // 16):
                zero_v[r, pl.ds(l * 16, 16)] = zero16
            return carry

        lax.fori_loop(0, zrows, zbody, 0)
        row0 = sid * rows_per_tile
        for p in range(rows_per_tile // zrows):
            pltpu.sync_copy(zero_v.at[pl.ds(0, zrows)],
                            agg_sh.at[pl.ds(row0 + p * zrows, zrows)])

        pltpu.sync_copy(srcw_hbm.at[sid], sidx_v)
        x0 = sid * (n_src // NS)
        pltpu.sync_copy(x_hbm.at[pl.ds(x0, n_src // NS), cid],
                        x_sh.at[pl.ds(x0, n_src // NS)])
        plsc.subcore_barrier()

        base0 = sid * ept
        bufs = ((rows_a, msg_a, didx_a, gsem_a, lsem_a, ssem_a, dsem_a),
                (rows_b, msg_b, didx_b, gsem_b, lsem_b, ssem_b, dsem_b))

        def issue_loads(j, rows_r, msg_r, didx_r, gsem, lsem, dsem):
            base = pl.multiple_of(base0 + j * CH, 8)
            pltpu.async_copy(lin_hbm.at[cid, pl.ds(base, CH)], msg_r, lsem)
            pltpu.async_copy(x_sh.at[sidx_v.at[j]], rows_r, gsem)
            pltpu.async_copy(dstw_hbm.at[sid, j], didx_r, dsem)

        def wait_loads(rows_r, msg_r, didx_r, gsem, lsem, dsem):
            pltpu.make_async_copy(lin_hbm.at[cid, pl.ds(0, CH)], msg_r,
                                  lsem).wait()
            pltpu.make_async_copy(x_sh.at[sidx_v.at[0]], rows_r,
                                  gsem).wait()
            pltpu.make_async_copy(dstw_hbm.at[sid, 0], didx_r, dsem).wait()

        def wait_scatter(rows_r, ssem):
            pltpu.make_async_copy(rows_r, agg_sh.at[didx_a], ssem).wait()

        def compute_chunk(rows_c, msg_c):
            @plsc.parallel_loop(0, CH, step=1, unroll=2)
            def _(r):
                for l in range(DH // 16):
                    sl = pl.ds(l * 16, 16)
                    rows_c[r, sl] = jnp.maximum(
                        rows_c[r, sl] + msg_c[r, sl], 0.0)

        issue_loads(0, rows_a, msg_a, didx_a, gsem_a, lsem_a, dsem_a)
        issue_loads(1, rows_b, msg_b, didx_b, gsem_b, lsem_b, dsem_b)
        wait_loads(rows_a, msg_a, didx_a, gsem_a, lsem_a, dsem_a)
        compute_chunk(rows_a, msg_a)
        pltpu.async_copy(rows_a, agg_sh.at[didx_a], ssem_a, add=True)
        wait_scatter(rows_a, ssem_a)
        issue_loads(2, rows_a, msg_a, didx_a, gsem_a, lsem_a, dsem_a)
        wait_loads(rows_b, msg_b, didx_b, gsem_b, lsem_b, dsem_b)
        compute_chunk(rows_b, msg_b)
        pltpu.async_copy(rows_b, agg_sh.at[didx_b], ssem_b, add=True)

        def pair_body(p, carry):
            for b in (0, 1):
                j = 2 * p + b
                rows_c, msg_c, didx_c, gsem_c, lsem_c, ssem_c, dsem_c = \
                    bufs[b]
                rows_n, msg_n, didx_n, gsem_n, lsem_n, ssem_n, dsem_n = \
                    bufs[1 - b]
                wait_scatter(rows_n, ssem_n)
                jn = jnp.where(j + 1 < nchunk, j + 1, 0)
                issue_loads(jn, rows_n, msg_n, didx_n, gsem_n, lsem_n,
                            dsem_n)
                wait_loads(rows_c, msg_c, didx_c, gsem_c, lsem_c, dsem_c)
                compute_chunk(rows_c, msg_c)
                pltpu.async_copy(rows_c, agg_sh.at[didx_c], ssem_c,
                                 add=True)
            return carry

        lax.fori_loop(1, nchunk // 2, pair_body, 0)
        wait_loads(rows_a, msg_a, didx_a, gsem_a, lsem_a, dsem_a)
        wait_scatter(rows_b, ssem_b)
        plsc.subcore_barrier()
        pltpu.sync_copy(agg_sh.at[pl.ds(row0, rows_per_tile)],
                        out_hbm.at[cid, pl.ds(row0, rows_per_tile)])

    return sc_msg


def kernel(x_author, x_paper, edge_index_a2p, edge_index_p2a,
           edge_attr_a2p, edge_attr_p2a, W_edge, b_edge, W_nn, b_nn, eps):
    n_a, _ = x_author.shape
    n_p, _ = x_paper.shape
    E = edge_index_a2p.shape[1]
    ept = E // NS
    nchunk = ept // CH

    def slab(ei):
        return (ei[0].reshape(NS, nchunk, CH), ei[1].reshape(NS, nchunk, CH))

    src_a2p, dst_a2p = slab(edge_index_a2p)
    src_p2a, dst_p2a = slab(edge_index_p2a)

    DE = W_edge.shape[0]
    zeros = jnp.zeros((DE, DH), jnp.float32)
    W_bd = jnp.stack([
        jnp.block([[W_edge[:, :DH], zeros], [zeros, W_edge[:, :DH]]]),
        jnp.block([[W_edge[:, DH:], zeros], [zeros, W_edge[:, DH:]]]),
    ], axis=0)
    b_pair = jnp.stack([
        jnp.concatenate([b_edge[:DH], b_edge[:DH]]),
        jnp.concatenate([b_edge[DH:], b_edge[DH:]]),
    ], axis=0).reshape(NC, 1, D)
    ea2_a2p = edge_attr_a2p.reshape(E // 2, 2 * DE)
    ea2_p2a = edge_attr_p2a.reshape(E // 2, 2 * DE)
    lin_a2p = _edge_lin(ea2_a2p, W_bd, b_pair)
    lin_p2a = _edge_lin(ea2_p2a, W_bd, b_pair)

    if n_a == n_p:
        sc_a2p = sc_p2a = _make_sc_msg(n_a, n_p, E)
    else:
        sc_a2p = _make_sc_msg(n_a, n_p, E)
        sc_p2a = _make_sc_msg(n_p, n_a, E)

    scale = jnp.reshape(1.0 + eps, (1,))

    xa, xp = x_author, x_paper
    for i in range(2):
        agg_p = sc_a2p(xa.reshape(n_a, NC, DH), src_a2p, dst_a2p,
                       lin_a2p.reshape(NC, E, DH))
        agg_a = sc_p2a(xp.reshape(n_p, NC, DH), src_p2a, dst_p2a,
                       lin_p2a.reshape(NC, E, DH))
        do_relu = i == 0
        new_p = _node_update(xp, agg_p, W_nn, b_nn, scale, do_relu)
        new_a = _node_update(xa, agg_a, W_nn, b_nn, scale, do_relu)
        xa, xp = new_a, new_p
    return (xa, xp)


# speedup vs baseline: 1.0796x; 1.0764x over previous
import functools

import jax
import jax.numpy as jnp
from jax import lax
from jax.experimental import pallas as pl
from jax.experimental.pallas import tpu as pltpu
from jax.experimental.pallas import tpu_sc as plsc

D = 128
DH = D // 2
NC, NS = 2, 16
CH = 80


def _edge_lin_body(ea_ref, w_ref, b_ref, out_ref):
    out_ref[0] = (
        jnp.dot(ea_ref[...], w_ref[0], preferred_element_type=jnp.float32)
        + b_ref[0]
    )


def _edge_lin(ea2, W_bd, b_pair):
    E2, DE2 = ea2.shape
    BE = 1280
    return pl.pallas_call(
        _edge_lin_body,
        grid=(NC, E2 // BE),
        in_specs=[
            pl.BlockSpec((BE, DE2), lambda j, i: (i, 0)),
            pl.BlockSpec((1, DE2, D), lambda j, i: (j, 0, 0)),
            pl.BlockSpec((1, 1, D), lambda j, i: (j, 0, 0)),
        ],
        out_specs=pl.BlockSpec((1, BE, D), lambda j, i: (j, i, 0)),
        out_shape=jax.ShapeDtypeStruct((NC, E2, D), jnp.float32),
    )(ea2, W_bd, b_pair)


def _update_body(s_ref, x_ref, a_ref, w_ref, b_ref, out_ref, *, do_relu):
    h = x_ref[...] * s_ref[0] + jnp.concatenate([a_ref[0], a_ref[1]], axis=1)
    o = jnp.dot(h, w_ref[...], preferred_element_type=jnp.float32) + b_ref[...]
    if do_relu:
        o = jnp.maximum(o, 0.0)
    out_ref[...] = o


def _node_update(x, agg, W_nn, b_nn, scale, do_relu):
    N = x.shape[0]
    BN = 2000
    return pl.pallas_call(
        functools.partial(_update_body, do_relu=do_relu),
        grid=(N // BN,),
        in_specs=[
            pl.BlockSpec(memory_space=pltpu.SMEM),
            pl.BlockSpec((BN, D), lambda i: (i, 0)),
            pl.BlockSpec((NC, BN, DH), lambda i: (0, i, 0)),
            pl.BlockSpec((D, D), lambda i: (0, 0)),
            pl.BlockSpec((1, D), lambda i: (0, 0)),
        ],
        out_specs=pl.BlockSpec((BN, D), lambda i: (i, 0)),
        out_shape=jax.ShapeDtypeStruct((N, D), jnp.float32),
    )(scale, x, agg, W_nn, b_nn.reshape(1, D))


def _pad_dst(n_dst):
    per_tile = -(-n_dst // NS)
    per_tile = -(-per_tile // CH) * CH
    return NS * per_tile


def _make_sc_msg(n_src, n_dst, E):
    ept = E // NS
    nchunk = ept // CH
    n_pad = _pad_dst(n_dst)
    rows_per_tile = n_pad // NS
    n80 = rows_per_tile // CH

    @functools.partial(
        pl.kernel,
        out_type=jax.ShapeDtypeStruct((NC, n_pad, DH), jnp.float32),
        mesh=plsc.VectorSubcoreMesh(
            core_axis_name="c", subcore_axis_name="s",
            num_cores=NC, num_subcores=NS,
        ),
        scratch_types=[
            pltpu.VMEM((nchunk, CH), jnp.int32),
            pltpu.VMEM((nchunk, CH), jnp.int32),
            pltpu.VMEM((CH, DH), jnp.float32),
            pltpu.VMEM((CH // 2, D), jnp.float32),
            pltpu.VMEM((CH, DH), jnp.float32),
            pltpu.VMEM((CH // 2, D), jnp.float32),
            pltpu.VMEM((CH, DH), jnp.float32),
            pltpu.VMEM_SHARED((n_pad, DH), jnp.float32),
            pltpu.SemaphoreType.DMA,
            pltpu.SemaphoreType.DMA,
            pltpu.SemaphoreType.DMA,
            pltpu.SemaphoreType.DMA,
            pltpu.SemaphoreType.DMA,
            pltpu.SemaphoreType.DMA,
        ],
        compiler_params=pltpu.CompilerParams(use_tc_tiling_on_sc=False),
    )
    def sc_msg(x_hbm, srcw_hbm, dstw_hbm, lin_hbm, out_hbm,
               sidx_v, didx_v, rows_a, msg_a, rows_b, msg_b, zero_v,
               agg_sh, gsem_a, lsem_a, ssem_a, gsem_b, lsem_b, ssem_b):
        cid = lax.axis_index("c")
        sid = lax.axis_index("s")

        zero16 = jnp.zeros((16,), jnp.float32)

        def zbody(r, carry):
            for l in range(DH // 16):
                zero_v[r, pl.ds(l * 16, 16)] = zero16
            return carry

        lax.fori_loop(0, CH, zbody, 0)
        row0 = sid * rows_per_tile
        for p in range(n80):
            pltpu.sync_copy(zero_v.at[pl.ds(0, CH)],
                            agg_sh.at[pl.ds(row0 + p * CH, CH)])

        pltpu.sync_copy(srcw_hbm.at[sid], sidx_v)
        pltpu.sync_copy(dstw_hbm.at[sid], didx_v)
        off16 = jnp.full((16,), cid, jnp.int32)

        def ibody(r, carry):
            for l in range(CH // 16):
                sl = pl.ds(l * 16, 16)
                sidx_v[r, sl] = sidx_v[r, sl] * 2 + off16
            return carry

        lax.fori_loop(0, nchunk, ibody, 0)
        plsc.subcore_barrier()

        base0 = sid * ept
        bufs = ((rows_a, msg_a, gsem_a, lsem_a, ssem_a),
                (rows_b, msg_b, gsem_b, lsem_b, ssem_b))

        def issue_loads(j, rows_r, msg_r, gsem, lsem):
            base = pl.multiple_of((base0 + j * CH) // 2, 8)
            pltpu.async_copy(lin_hbm.at[cid, pl.ds(base, CH // 2)], msg_r,
                             lsem)
            pltpu.async_copy(x_hbm.at[sidx_v.at[j]], rows_r, gsem)

        def wait_loads(rows_r, msg_r, gsem, lsem):
            pltpu.make_async_copy(lin_hbm.at[cid, pl.ds(0, CH // 2)], msg_r,
                                  lsem).wait()
            pltpu.make_async_copy(x_hbm.at[sidx_v.at[0]], rows_r,
                                  gsem).wait()

        def wait_scatter(rows_r, ssem):
            pltpu.make_async_copy(rows_r, agg_sh.at[didx_v.at[0]],
                                  ssem).wait()

        def compute_chunk(rows_c, msg_c):
            def cbody(r, c2):
                for l in range(D // 16):
                    e = 2 * r + l // (DH // 16)
                    sl = pl.ds((l % (DH // 16)) * 16, 16)
                    rows_c[e, sl] = jnp.maximum(
                        rows_c[e, sl] + msg_c[r, pl.ds(l * 16, 16)], 0.0)
                return c2

            lax.fori_loop(0, CH // 2, cbody, 0)

        issue_loads(0, rows_a, msg_a, gsem_a, lsem_a)
        issue_loads(1, rows_b, msg_b, gsem_b, lsem_b)
        wait_loads(rows_a, msg_a, gsem_a, lsem_a)
        compute_chunk(rows_a, msg_a)
        pltpu.async_copy(rows_a, agg_sh.at[didx_v.at[0]], ssem_a, add=True)
        wait_scatter(rows_a, ssem_a)
        issue_loads(2, rows_a, msg_a, gsem_a, lsem_a)
        wait_loads(rows_b, msg_b, gsem_b, lsem_b)
        compute_chunk(rows_b, msg_b)
        pltpu.async_copy(rows_b, agg_sh.at[didx_v.at[1]], ssem_b, add=True)

        def pair_body(p, carry):
            for b in (0, 1):
                j = 2 * p + b
                rows_c, msg_c, gsem_c, lsem_c, ssem_c = bufs[b]
                rows_n, msg_n, gsem_n, lsem_n, ssem_n = bufs[1 - b]
                wait_scatter(rows_n, ssem_n)
                jn = jnp.where(j + 1 < nchunk, j + 1, 0)
                issue_loads(jn, rows_n, msg_n, gsem_n, lsem_n)
                wait_loads(rows_c, msg_c, gsem_c, lsem_c)
                compute_chunk(rows_c, msg_c)
                pltpu.async_copy(rows_c, agg_sh.at[didx_v.at[j]], ssem_c,
                                 add=True)
            return carry

        lax.fori_loop(1, nchunk // 2, pair_body, 0)
        wait_loads(rows_a, msg_a, gsem_a, lsem_a)
        wait_scatter(rows_b, ssem_b)
        plsc.subcore_barrier()
        pltpu.sync_copy(agg_sh.at[pl.ds(row0, rows_per_tile)],
                        out_hbm.at[cid, pl.ds(row0, rows_per_tile)])

    return sc_msg


def kernel(x_author, x_paper, edge_index_a2p, edge_index_p2a,
           edge_attr_a2p, edge_attr_p2a, W_edge, b_edge, W_nn, b_nn, eps):
    n_a, _ = x_author.shape
    n_p, _ = x_paper.shape
    E = edge_index_a2p.shape[1]
    ept = E // NS
    nchunk = ept // CH

    def slab(ei):
        return (ei[0].reshape(NS, nchunk, CH), ei[1].reshape(NS, nchunk, CH))

    src_a2p, dst_a2p = slab(edge_index_a2p)
    src_p2a, dst_p2a = slab(edge_index_p2a)

    DE = W_edge.shape[0]
    zeros = jnp.zeros((DE, DH), jnp.float32)
    W_bd = jnp.stack([
        jnp.block([[W_edge[:, :DH], zeros], [zeros, W_edge[:, :DH]]]),
        jnp.block([[W_edge[:, DH:], zeros], [zeros, W_edge[:, DH:]]]),
    ], axis=0)
    b_pair = jnp.stack([
        jnp.concatenate([b_edge[:DH], b_edge[:DH]]),
        jnp.concatenate([b_edge[DH:], b_edge[DH:]]),
    ], axis=0).reshape(NC, 1, D)
    ea2_a2p = edge_attr_a2p.reshape(E // 2, 2 * DE)
    ea2_p2a = edge_attr_p2a.reshape(E // 2, 2 * DE)
    lin_a2p = _edge_lin(ea2_a2p, W_bd, b_pair)
    lin_p2a = _edge_lin(ea2_p2a, W_bd, b_pair)

    if n_a == n_p:
        sc_a2p = sc_p2a = _make_sc_msg(n_a, n_p, E)
    else:
        sc_a2p = _make_sc_msg(n_a, n_p, E)
        sc_p2a = _make_sc_msg(n_p, n_a, E)

    scale = jnp.reshape(1.0 + eps, (1,))

    xa, xp = x_author, x_paper
    for i in range(2):
        agg_p = sc_a2p(xa.reshape(NC * n_a, DH), src_a2p, dst_a2p, lin_a2p)
        agg_a = sc_p2a(xp.reshape(NC * n_p, DH), src_p2a, dst_p2a, lin_p2a)
        do_relu = i == 0
        new_p = _node_update(xp, agg_p, W_nn, b_nn, scale, do_relu)
        new_a = _node_update(xa, agg_a, W_nn, b_nn, scale, do_relu)
        xa, xp = new_a, new_p
    return (xa, xp)


# speedup vs baseline: 1.2598x; 1.1669x over previous
import functools

import jax
import jax.numpy as jnp
from jax import lax
from jax.experimental import pallas as pl
from jax.experimental.pallas import tpu as pltpu
from jax.experimental.pallas import tpu_sc as plsc

D = 128
DH = D // 2
NC, NS = 2, 16
CH = 80


def _edge_lin_body(ea_ref, w_ref, b_ref, out_ref):
    ea = ea_ref[...]
    for c in range(NC):
        out_ref[c] = (
            jnp.dot(ea, w_ref[c], preferred_element_type=jnp.float32)
            + b_ref[c]
        )


def _edge_lin(ea2, W_bd, b_pair):
    E2, DE2 = ea2.shape
    BE = 3200
    return pl.pallas_call(
        _edge_lin_body,
        grid=(E2 // BE,),
        in_specs=[
            pl.BlockSpec((BE, DE2), lambda i: (i, 0)),
            pl.BlockSpec((NC, DE2, D), lambda i: (0, 0, 0)),
            pl.BlockSpec((NC, 1, D), lambda i: (0, 0, 0)),
        ],
        out_specs=pl.BlockSpec((NC, BE, D), lambda i: (0, i, 0)),
        out_shape=jax.ShapeDtypeStruct((NC, E2, D), jnp.float32),
    )(ea2, W_bd, b_pair)


def _update_body(s_ref, x_ref, a_ref, w_ref, b_ref, out_ref, *, do_relu):
    h = x_ref[...] * s_ref[0] + jnp.concatenate([a_ref[0], a_ref[1]], axis=1)
    o = jnp.dot(h, w_ref[...], preferred_element_type=jnp.float32) + b_ref[...]
    if do_relu:
        o = jnp.maximum(o, 0.0)
    out_ref[...] = o


def _node_update(x, agg, W_nn, b_nn, scale, do_relu):
    N = x.shape[0]
    BN = 2000
    return pl.pallas_call(
        functools.partial(_update_body, do_relu=do_relu),
        grid=(N // BN,),
        in_specs=[
            pl.BlockSpec(memory_space=pltpu.SMEM),
            pl.BlockSpec((BN, D), lambda i: (i, 0)),
            pl.BlockSpec((NC, BN, DH), lambda i: (0, i, 0)),
            pl.BlockSpec((D, D), lambda i: (0, 0)),
            pl.BlockSpec((1, D), lambda i: (0, 0)),
        ],
        out_specs=pl.BlockSpec((BN, D), lambda i: (i, 0)),
        out_shape=jax.ShapeDtypeStruct((N, D), jnp.float32),
    )(scale, x, agg, W_nn, b_nn.reshape(1, D))


def _pad_dst(n_dst):
    per_tile = -(-n_dst // NS)
    per_tile = -(-per_tile // CH) * CH
    return NS * per_tile


def _make_sc_msg(n_src, n_dst, E):
    ept = E // NS
    nchunk = ept // CH
    n_pad = _pad_dst(n_dst)
    rows_per_tile = n_pad // NS
    n80 = rows_per_tile // CH

    @functools.partial(
        pl.kernel,
        out_type=jax.ShapeDtypeStruct((NC, n_pad, DH), jnp.float32),
        mesh=plsc.VectorSubcoreMesh(
            core_axis_name="c", subcore_axis_name="s",
            num_cores=NC, num_subcores=NS,
        ),
        scratch_types=[
            pltpu.VMEM((nchunk, CH), jnp.int32),
            pltpu.VMEM((nchunk, CH), jnp.int32),
            pltpu.VMEM((CH, DH), jnp.float32),
            pltpu.VMEM((CH // 2, D), jnp.float32),
            pltpu.VMEM((CH, DH), jnp.float32),
            pltpu.VMEM((CH // 2, D), jnp.float32),
            pltpu.VMEM((CH, DH), jnp.float32),
            pltpu.VMEM_SHARED((n_pad, DH), jnp.float32),
            pltpu.SemaphoreType.DMA,
            pltpu.SemaphoreType.DMA,
            pltpu.SemaphoreType.DMA,
            pltpu.SemaphoreType.DMA,
            pltpu.SemaphoreType.DMA,
            pltpu.SemaphoreType.DMA,
        ],
        compiler_params=pltpu.CompilerParams(use_tc_tiling_on_sc=False),
    )
    def sc_msg(x_hbm, srcw_hbm, dstw_hbm, lin_hbm, out_hbm,
               sidx_v, didx_v, rows_a, msg_a, rows_b, msg_b, zero_v,
               agg_sh, gsem_a, lsem_a, ssem_a, gsem_b, lsem_b, ssem_b):
        cid = lax.axis_index("c")
        sid = lax.axis_index("s")

        zero16 = jnp.zeros((16,), jnp.float32)

        def zbody(r, carry):
            for l in range(DH // 16):
                zero_v[r, pl.ds(l * 16, 16)] = zero16
            return carry

        lax.fori_loop(0, CH, zbody, 0)
        row0 = sid * rows_per_tile
        for p in range(n80):
            pltpu.sync_copy(zero_v.at[pl.ds(0, CH)],
                            agg_sh.at[pl.ds(row0 + p * CH, CH)])

        pltpu.sync_copy(srcw_hbm.at[sid], sidx_v)
        pltpu.sync_copy(dstw_hbm.at[sid], didx_v)
        off16 = jnp.full((16,), cid, jnp.int32)

        def ibody(r, carry):
            for l in range(CH // 16):
                sl = pl.ds(l * 16, 16)
                sidx_v[r, sl] = sidx_v[r, sl] * 2 + off16
            return carry

        lax.fori_loop(0, nchunk, ibody, 0)
        plsc.subcore_barrier()

        base0 = sid * ept
        bufs = ((rows_a, msg_a, gsem_a, lsem_a, ssem_a),
                (rows_b, msg_b, gsem_b, lsem_b, ssem_b))

        def issue_loads(j, rows_r, msg_r, gsem, lsem):
            base = pl.multiple_of((base0 + j * CH) // 2, 8)
            pltpu.async_copy(lin_hbm.at[cid, pl.ds(base, CH // 2)], msg_r,
                             lsem)
            pltpu.async_copy(x_hbm.at[sidx_v.at[j]], rows_r, gsem)

        def wait_loads(rows_r, msg_r, gsem, lsem):
            pltpu.make_async_copy(lin_hbm.at[cid, pl.ds(0, CH // 2)], msg_r,
                                  lsem).wait()
            pltpu.make_async_copy(x_hbm.at[sidx_v.at[0]], rows_r,
                                  gsem).wait()

        def wait_scatter(rows_r, ssem):
            pltpu.make_async_copy(rows_r, agg_sh.at[didx_v.at[0]],
                                  ssem).wait()

        def compute_chunk(rows_c, msg_c):
            def cbody(r, c2):
                for l in range(D // 16):
                    e = 2 * r + l // (DH // 16)
                    sl = pl.ds((l % (DH // 16)) * 16, 16)
                    rows_c[e, sl] = jnp.maximum(
                        rows_c[e, sl] + msg_c[r, pl.ds(l * 16, 16)], 0.0)
                return c2

            lax.fori_loop(0, CH // 2, cbody, 0)

        issue_loads(0, rows_a, msg_a, gsem_a, lsem_a)
        issue_loads(1, rows_b, msg_b, gsem_b, lsem_b)
        wait_loads(rows_a, msg_a, gsem_a, lsem_a)
        compute_chunk(rows_a, msg_a)
        pltpu.async_copy(rows_a, agg_sh.at[didx_v.at[0]], ssem_a, add=True)
        wait_scatter(rows_a, ssem_a)
        issue_loads(2, rows_a, msg_a, gsem_a, lsem_a)
        wait_loads(rows_b, msg_b, gsem_b, lsem_b)
        compute_chunk(rows_b, msg_b)
        pltpu.async_copy(rows_b, agg_sh.at[didx_v.at[1]], ssem_b, add=True)

        def pair_body(p, carry):
            for b in (0, 1):
                j = 2 * p + b
                rows_c, msg_c, gsem_c, lsem_c, ssem_c = bufs[b]
                rows_n, msg_n, gsem_n, lsem_n, ssem_n = bufs[1 - b]
                wait_scatter(rows_n, ssem_n)
                jn = jnp.where(j + 1 < nchunk, j + 1, 0)
                issue_loads(jn, rows_n, msg_n, gsem_n, lsem_n)
                wait_loads(rows_c, msg_c, gsem_c, lsem_c)
                compute_chunk(rows_c, msg_c)
                pltpu.async_copy(rows_c, agg_sh.at[didx_v.at[j]], ssem_c,
                                 add=True)
            return carry

        lax.fori_loop(1, nchunk // 2, pair_body, 0)
        wait_loads(rows_a, msg_a, gsem_a, lsem_a)
        wait_scatter(rows_b, ssem_b)
        plsc.subcore_barrier()
        pltpu.sync_copy(agg_sh.at[pl.ds(row0, rows_per_tile)],
                        out_hbm.at[cid, pl.ds(row0, rows_per_tile)])

    return sc_msg


def kernel(x_author, x_paper, edge_index_a2p, edge_index_p2a,
           edge_attr_a2p, edge_attr_p2a, W_edge, b_edge, W_nn, b_nn, eps):
    n_a, _ = x_author.shape
    n_p, _ = x_paper.shape
    E = edge_index_a2p.shape[1]
    ept = E // NS
    nchunk = ept // CH

    def slab(ei):
        return (ei[0].reshape(NS, nchunk, CH), ei[1].reshape(NS, nchunk, CH))

    src_a2p, dst_a2p = slab(edge_index_a2p)
    src_p2a, dst_p2a = slab(edge_index_p2a)

    DE = W_edge.shape[0]
    zeros = jnp.zeros((DE, DH), jnp.float32)
    W_bd = jnp.stack([
        jnp.block([[W_edge[:, :DH], zeros], [zeros, W_edge[:, :DH]]]),
        jnp.block([[W_edge[:, DH:], zeros], [zeros, W_edge[:, DH:]]]),
    ], axis=0)
    b_pair = jnp.stack([
        jnp.concatenate([b_edge[:DH], b_edge[:DH]]),
        jnp.concatenate([b_edge[DH:], b_edge[DH:]]),
    ], axis=0).reshape(NC, 1, D)
    ea2_a2p = edge_attr_a2p.reshape(E // 2, 2 * DE)
    ea2_p2a = edge_attr_p2a.reshape(E // 2, 2 * DE)
    lin_a2p = _edge_lin(ea2_a2p, W_bd, b_pair)
    lin_p2a = _edge_lin(ea2_p2a, W_bd, b_pair)

    if n_a == n_p:
        sc_a2p = sc_p2a = _make_sc_msg(n_a, n_p, E)
    else:
        sc_a2p = _make_sc_msg(n_a, n_p, E)
        sc_p2a = _make_sc_msg(n_p, n_a, E)

    scale = jnp.reshape(1.0 + eps, (1,))

    xa, xp = x_author, x_paper
    for i in range(2):
        agg_p = sc_a2p(xa.reshape(NC * n_a, DH), src_a2p, dst_a2p, lin_a2p)
        agg_a = sc_p2a(xp.reshape(NC * n_p, DH), src_p2a, dst_p2a, lin_p2a)
        do_relu = i == 0
        new_p = _node_update(xp, agg_p, W_nn, b_nn, scale, do_relu)
        new_a = _node_update(xa, agg_a, W_nn, b_nn, scale, do_relu)
        xa, xp = new_a, new_p
    return (xa, xp)
